# baseline (speedup 1.0000x reference)
import functools

import jax
import jax.numpy as jnp
from jax import lax
from jax.experimental import pallas as pl
from jax.experimental.pallas import tpu as pltpu
from jax.experimental.pallas import tpu_sc as plsc

N_NODES = 10000
D = 128
BM = 1000

NC = 2
NS = 16
NW = NC * NS
E = 320000
EPW = E // NW
CBH = 64
EPWP = 10240
PAIRS = EPWP // (2 * CBH)
RPS = 624
TAIL = N_NODES - NS * RPS


def _mm_body(x_ref, w_ref, o_ref):
    o_ref[...] = jnp.dot(x_ref[...], w_ref[...], preferred_element_type=jnp.float32)


def _mm(x, wc):
    m, k = x.shape
    n = wc.shape[1]
    return pl.pallas_call(
        _mm_body,
        grid=(m // BM,),
        in_specs=[
            pl.BlockSpec((BM, k), lambda i: (i, 0)),
            pl.BlockSpec((k, n), lambda i: (0, 0)),
        ],
        out_specs=pl.BlockSpec((BM, n), lambda i: (i, 0)),
        out_shape=jax.ShapeDtypeStruct((m, n), jnp.float32),
    )(x, wc)


def _norm_mm_body(a_ref, d_ref, b_ref, w_ref, o_ref):
    den = jnp.sum(d_ref[...], axis=1) + 1e-16
    g = (a_ref[0] + a_ref[1]) / den[:, None] + b_ref[...]
    g = jnp.maximum(g, 0.0)
    o_ref[...] = jnp.dot(g, w_ref[...], preferred_element_type=jnp.float32)


def _norm_mm(acc2, den32, b, wc):
    m = acc2.shape[1]
    n = wc.shape[1]
    return pl.pallas_call(
        _norm_mm_body,
        grid=(m // BM,),
        in_specs=[
            pl.BlockSpec((2, BM, D), lambda i: (0, i, 0)),
            pl.BlockSpec((BM, NW), lambda i: (i, 0)),
            pl.BlockSpec((1, D), lambda i: (0, 0)),
            pl.BlockSpec((D, n), lambda i: (0, 0)),
        ],
        out_specs=pl.BlockSpec((BM, n), lambda i: (i, 0)),
        out_shape=jax.ShapeDtypeStruct((m, n), jnp.float32),
    )(acc2, den32.T, b.reshape(1, D), wc)


def _norm_body(a_ref, d_ref, b_ref, o_ref):
    den = jnp.sum(d_ref[...], axis=1) + 1e-16
    o_ref[...] = (a_ref[0] + a_ref[1]) / den[:, None] + b_ref[...]


def _norm(acc2, den32, b):
    m = acc2.shape[1]
    return pl.pallas_call(
        _norm_body,
        grid=(m // BM,),
        in_specs=[
            pl.BlockSpec((2, BM, D), lambda i: (0, i, 0)),
            pl.BlockSpec((BM, NW), lambda i: (i, 0)),
            pl.BlockSpec((1, D), lambda i: (0, 0)),
        ],
        out_specs=pl.BlockSpec((BM, D), lambda i: (i, 0)),
        out_shape=jax.ShapeDtypeStruct((m, D), jnp.float32),
    )(acc2, den32.T, b.reshape(1, D))


_SC_MESH = dict(core_axis_name="c", subcore_axis_name="s", num_cores=NC,
                num_subcores=NS)


def _attn_sc(a_src, a_dst, srcf, dstf):
    @functools.partial(
        pl.kernel,
        out_type=(
            jax.ShapeDtypeStruct((NW, 1, EPWP), jnp.float32),
            jax.ShapeDtypeStruct((NW, 1, N_NODES), jnp.float32),
        ),
        mesh=plsc.VectorSubcoreMesh(**_SC_MESH),
        scratch_types=dict(
            src1=pltpu.VMEM((EPWP,), jnp.int32),
            dst1=pltpu.VMEM((EPWP,), jnp.int32),
            asv=pltpu.VMEM((N_NODES,), jnp.float32),
            adv=pltpu.VMEM((N_NODES,), jnp.float32),
            wv=pltpu.VMEM((EPWP,), jnp.float32),
            denv=pltpu.VMEM((N_NODES,), jnp.float32),
        ),
        compiler_params=pltpu.CompilerParams(needs_layout_passes=False),
    )
    def k(asrc_hbm, adst_hbm, src_hbm, dst_hbm, w_out, den_out,
          src1, dst1, asv, adv, wv, denv):
        cid = lax.axis_index("c")
        sid = lax.axis_index("s")
        wid = sid * NC + cid

        pltpu.sync_copy(src_hbm.at[wid, 0], src1)
        pltpu.sync_copy(dst_hbm.at[wid, 0], dst1)
        pltpu.sync_copy(asrc_hbm, asv)
        pltpu.sync_copy(adst_hbm, adv)

        zeros16 = jnp.zeros((16,), jnp.float32)

        def zden(i, _):
            denv[pl.ds(i * 16, 16)] = zeros16
            return 0
        lax.fori_loop(0, N_NODES // 16, zden, 0)

        def edge16(j, _):
            sv = src1[pl.ds(j * 16, 16)]
            dv = dst1[pl.ds(j * 16, 16)]
            a = plsc.load_gather(asv, [sv]) + plsc.load_gather(adv, [dv])
            a = jnp.where(a > 0, a, a * 0.2)
            w = jnp.exp(a)
            wv[pl.ds(j * 16, 16)] = w
            plsc.addupdate_scatter(denv, [dv], w)
            return 0
        lax.fori_loop(0, EPW // 16, edge16, 0)

        for t in range((EPWP - EPW) // 16):
            wv[pl.ds(EPW + t * 16, 16)] = zeros16

        pltpu.sync_copy(wv, w_out.at[wid, 0])
        pltpu.sync_copy(denv, den_out.at[wid, 0])

    w3, den = k(a_src, a_dst, srcf, dstf)
    return w3, den.reshape(NW, N_NODES)


def _agg_sc(h, w4, srcf, dst4):
    @functools.partial(
        pl.kernel,
        out_type=jax.ShapeDtypeStruct((NC, N_NODES, D), jnp.float32),
        mesh=plsc.VectorSubcoreMesh(**_SC_MESH),
        scratch_types=dict(
            src1=pltpu.VMEM((EPWP,), jnp.int32),
            dbA=pltpu.VMEM((2, CBH), jnp.int32),
            wbA=pltpu.VMEM((2, CBH), jnp.float32),
            dbB=pltpu.VMEM((2, CBH), jnp.int32),
            wbB=pltpu.VMEM((2, CBH), jnp.float32),
            r0=pltpu.VMEM((CBH, D // 2), jnp.float32),
            r1=pltpu.VMEM((CBH, D // 2), jnp.float32),
            ro0=pltpu.VMEM((CBH, D), jnp.float32),
            ro1=pltpu.VMEM((CBH, D), jnp.float32),
            acc_s=pltpu.VMEM_SHARED((N_NODES, D), jnp.float32),
            sdwA=pltpu.SemaphoreType.DMA,
            sdwB=pltpu.SemaphoreType.DMA,
            sg0=pltpu.SemaphoreType.DMA,
            sg1=pltpu.SemaphoreType.DMA,
            ss0=pltpu.SemaphoreType.DMA,
            ss1=pltpu.SemaphoreType.DMA,
        ),
        compiler_params=pltpu.CompilerParams(needs_layout_passes=False,
                                             use_tc_tiling_on_sc=False),
    )
    def k(h_hbm, w_hbm, src_hbm, dst_hbm, acc_out,
          src1, dbA, wbA, dbB, wbB, r0, r1, ro0, ro1, acc_s,
          sdwA, sdwB, sg0, sg1, ss0, ss1):
        cid = lax.axis_index("c")
        sid = lax.axis_index("s")
        wid = sid * NC + cid

        pltpu.sync_copy(src_hbm.at[wid, 0], src1)

        zeros16 = jnp.zeros((16,), jnp.float32)

        for i in range(16):
            for t in range(D // 16):
                ro0[i, pl.ds(t * 16, 16)] = zeros16

        def zacc(i, _):
            pltpu.sync_copy(ro0.at[pl.ds(0, 16)],
                            acc_s.at[pl.ds(sid * RPS + i * 16, 16)])
            return 0
        lax.fori_loop(0, RPS // 16, zacc, 0)

        @pl.when(sid == NS - 1)
        def _():
            pltpu.sync_copy(ro0.at[pl.ds(0, 16)], acc_s.at[pl.ds(NS * RPS, TAIL)])

        plsc.subcore_barrier()

        def scale(rp, ro, wbuf, half):
            for g in range(CBH // 16):
                wv16 = wbuf[half, pl.ds(g * 16, 16)]
                for r in range(16):
                    ws = jnp.full((16,), wv16[r], jnp.float32)
                    row = g * 16 + r
                    for t in range(D // 32):
                        v = rp[row, pl.ds(t * 16, 16)]
                        a, b = plsc.unpack(plsc.bitcast(v, jnp.bfloat16),
                                           format=plsc.PackFormat.INTERLEAVED)
                        ro[row, pl.ds(t * 32, 16)] = a * ws
                        ro[row, pl.ds(t * 32 + 16, 16)] = b * ws

        def gather(edge0, rbuf, sem):
            pltpu.async_copy(h_hbm.at[src1.at[pl.ds(edge0, CBH)]], rbuf, sem)

        def wait_g(rbuf, sem):
            pltpu.make_async_copy(h_hbm.at[pl.ds(0, CBH)], rbuf, sem).wait()

        def wait_dw(dbuf, wbuf, sem):
            pltpu.make_async_copy(dst_hbm.at[wid, 0], dbuf, sem).wait()
            pltpu.make_async_copy(w_hbm.at[wid, 0], wbuf, sem).wait()

        def drain_scatter(robuf, sem):
            pltpu.make_async_copy(acc_out.at[0].at[pl.ds(0, CBH)], robuf,
                                  sem).wait()

        def issue_dw(pair, dbuf, wbuf, sem):
            pltpu.async_copy(dst_hbm.at[wid, pair], dbuf, sem)
            pltpu.async_copy(w_hbm.at[wid, pair], wbuf, sem)

        U = PAIRS // 2

        issue_dw(0, dbA, wbA, sdwA)
        gather(0, r0, sg0)
        gather(CBH, r1, sg1)

        def iter_body(u, _):
            e0 = u * 4 * CBH
            wait_g(r0, sg0)
            wait_dw(dbA, wbA, sdwA)

            @pl.when(u > 0)
            def _():
                drain_scatter(ro0, ss0)
            scale(r0, ro0, wbA, 0)
            pltpu.async_copy(ro0, acc_s.at[dbA.at[0]], ss0, add=True)

            wait_g(r1, sg1)

            @pl.when(u > 0)
            def _():
                drain_scatter(ro1, ss1)
            scale(r1, ro1, wbA, 1)
            pltpu.async_copy(ro1, acc_s.at[dbA.at[1]], ss1, add=True)

            issue_dw(2 * u + 1, dbB, wbB, sdwB)
            gather(e0 + 2 * CBH, r0, sg0)
            gather(e0 + 3 * CBH, r1, sg1)

            wait_g(r0, sg0)
            wait_dw(dbB, wbB, sdwB)
            drain_scatter(ro0, ss0)
            scale(r0, ro0, wbB, 0)
            pltpu.async_copy(ro0, acc_s.at[dbB.at[0]], ss0, add=True)

            wait_g(r1, sg1)
            drain_scatter(ro1, ss1)
            scale(r1, ro1, wbB, 1)
            pltpu.async_copy(ro1, acc_s.at[dbB.at[1]], ss1, add=True)

            @pl.when(u < U - 1)
            def _():
                issue_dw(2 * u + 2, dbA, wbA, sdwA)
                gather(e0 + 4 * CBH, r0, sg0)
                gather(e0 + 5 * CBH, r1, sg1)
            return 0
        lax.fori_loop(0, U, iter_body, 0)

        drain_scatter(ro0, ss0)
        drain_scatter(ro1, ss1)

        plsc.subcore_barrier()
        pltpu.sync_copy(acc_s.at[pl.ds(sid * RPS, RPS)],
                        acc_out.at[cid].at[pl.ds(sid * RPS, RPS)])

        @pl.when(sid == NS - 1)
        def _():
            pltpu.sync_copy(acc_s.at[pl.ds(NS * RPS, TAIL)],
                            acc_out.at[cid].at[pl.ds(NS * RPS, TAIL)])

    return k(h, w4, srcf, dst4)


def _pack_h(h):
    n = h.shape[0]
    hb = h.astype(jnp.bfloat16).reshape(n, D // 32, 2, 16)
    st = jnp.stack([hb[:, :, 0, :], hb[:, :, 1, :]], axis=-1)
    return lax.bitcast_convert_type(st, jnp.float32).reshape(n, D // 2)


def _edge_sc(h, a_src, a_dst, srcf, dstf, dst4):
    w3, den32 = _attn_sc(a_src, a_dst, srcf, dstf)
    w4 = w3.reshape(NW, PAIRS, 2, CBH)
    acc2 = _agg_sc(_pack_h(h), w4, srcf, dst4)
    return acc2, den32


def _augment(W, att_src, att_dst):
    A = jnp.zeros((D, D), jnp.float32)
    A = A.at[:, 0].set(att_src).at[:, 1].set(att_dst)
    return jnp.concatenate([W, W @ A], axis=1)


def kernel(x, edge_index, W1, att_src1, att_dst1, b1, W2, att_src2, att_dst2, b2):
    pad = ((0, 0), (0, EPWP - EPW))
    srcf = jnp.pad(edge_index[0].reshape(NW, EPW), pad).reshape(NW, 1, EPWP)
    dstf = jnp.pad(edge_index[1].reshape(NW, EPW), pad).reshape(NW, 1, EPWP)
    dst4 = dstf.reshape(NW, PAIRS, 2, CBH)

    wc1 = _augment(W1, att_src1, att_dst1)
    out1 = _mm(x, wc1)
    h1 = out1[:, :D]
    as1 = out1[:, D]
    ad1 = out1[:, D + 1]
    acc1, den1 = _edge_sc(h1, as1, ad1, srcf, dstf, dst4)

    wc2 = _augment(W2, att_src2, att_dst2)
    out2 = _norm_mm(acc1, den1, b1, wc2)
    h2 = out2[:, :D]
    as2 = out2[:, D]
    ad2 = out2[:, D + 1]
    acc2, den2 = _edge_sc(h2, as2, ad2, srcf, dstf, dst4)

    return _norm(acc2, den2, b2)


# speedup vs baseline: 1.4331x; 1.4331x over previous
import functools

import jax
import jax.numpy as jnp
from jax import lax
from jax.experimental import pallas as pl
from jax.experimental.pallas import tpu as pltpu
from jax.experimental.pallas import tpu_sc as plsc

N_NODES = 10000
D = 128
BM = 1000

NC = 2
NS = 16
NW = NC * NS
E = 320000
EPW = E // NW
CBH = 64
EPWP = 10112
PAIRS = EPWP // (2 * CBH)
RPS = 624
TAIL = N_NODES - NS * RPS


def _mm_body(x_ref, w_ref, o_ref):
    o_ref[...] = jnp.dot(x_ref[...], w_ref[...], preferred_element_type=jnp.float32)


def _mm(x, wc):
    m, k = x.shape
    n = wc.shape[1]
    return pl.pallas_call(
        _mm_body,
        grid=(m // BM,),
        in_specs=[
            pl.BlockSpec((BM, k), lambda i: (i, 0)),
            pl.BlockSpec((k, n), lambda i: (0, 0)),
        ],
        out_specs=pl.BlockSpec((BM, n), lambda i: (i, 0)),
        out_shape=jax.ShapeDtypeStruct((m, n), jnp.float32),
    )(x, wc)


def _norm_mm_body(a_ref, d_ref, b_ref, w_ref, o_ref):
    den = jnp.sum(d_ref[...], axis=1) + 1e-16
    g = (a_ref[0] + a_ref[1]) / den[:, None] + b_ref[...]
    g = jnp.maximum(g, 0.0)
    o_ref[...] = jnp.dot(g, w_ref[...], preferred_element_type=jnp.float32)


def _norm_mm(acc2, den32, b, wc):
    m = acc2.shape[1]
    n = wc.shape[1]
    return pl.pallas_call(
        _norm_mm_body,
        grid=(m // BM,),
        in_specs=[
            pl.BlockSpec((2, BM, D), lambda i: (0, i, 0)),
            pl.BlockSpec((BM, NW), lambda i: (i, 0)),
            pl.BlockSpec((1, D), lambda i: (0, 0)),
            pl.BlockSpec((D, n), lambda i: (0, 0)),
        ],
        out_specs=pl.BlockSpec((BM, n), lambda i: (i, 0)),
        out_shape=jax.ShapeDtypeStruct((m, n), jnp.float32),
    )(acc2, den32.T, b.reshape(1, D), wc)


def _norm_body(a_ref, d_ref, b_ref, o_ref):
    den = jnp.sum(d_ref[...], axis=1) + 1e-16
    o_ref[...] = (a_ref[0] + a_ref[1]) / den[:, None] + b_ref[...]


def _norm(acc2, den32, b):
    m = acc2.shape[1]
    return pl.pallas_call(
        _norm_body,
        grid=(m // BM,),
        in_specs=[
            pl.BlockSpec((2, BM, D), lambda i: (0, i, 0)),
            pl.BlockSpec((BM, NW), lambda i: (i, 0)),
            pl.BlockSpec((1, D), lambda i: (0, 0)),
        ],
        out_specs=pl.BlockSpec((BM, D), lambda i: (i, 0)),
        out_shape=jax.ShapeDtypeStruct((m, D), jnp.float32),
    )(acc2, den32.T, b.reshape(1, D))


_SC_MESH = dict(core_axis_name="c", subcore_axis_name="s", num_cores=NC,
                num_subcores=NS)


def _attn_sc(a_src, a_dst, srcf, dstf):
    @functools.partial(
        pl.kernel,
        out_type=(
            jax.ShapeDtypeStruct((NW, 1, EPWP), jnp.float32),
            jax.ShapeDtypeStruct((NW, 1, N_NODES), jnp.float32),
        ),
        mesh=plsc.VectorSubcoreMesh(**_SC_MESH),
        scratch_types=dict(
            src1=pltpu.VMEM((EPWP,), jnp.int32),
            dst1=pltpu.VMEM((EPWP,), jnp.int32),
            asv=pltpu.VMEM((N_NODES,), jnp.float32),
            adv=pltpu.VMEM((N_NODES,), jnp.float32),
            wv=pltpu.VMEM((EPWP,), jnp.float32),
            denv=pltpu.VMEM((N_NODES,), jnp.float32),
        ),
        compiler_params=pltpu.CompilerParams(needs_layout_passes=False),
    )
    def k(asrc_hbm, adst_hbm, src_hbm, dst_hbm, w_out, den_out,
          src1, dst1, asv, adv, wv, denv):
        cid = lax.axis_index("c")
        sid = lax.axis_index("s")
        wid = sid * NC + cid

        pltpu.sync_copy(src_hbm.at[wid, 0], src1)
        pltpu.sync_copy(dst_hbm.at[wid, 0], dst1)
        pltpu.sync_copy(asrc_hbm, asv)
        pltpu.sync_copy(adst_hbm, adv)

        zeros16 = jnp.zeros((16,), jnp.float32)

        def zden(i, _):
            denv[pl.ds(i * 16, 16)] = zeros16
            return 0
        lax.fori_loop(0, N_NODES // 16, zden, 0)

        def edge16(j, _):
            sv = src1[pl.ds(j * 16, 16)]
            dv = dst1[pl.ds(j * 16, 16)]
            a = plsc.load_gather(asv, [sv]) + plsc.load_gather(adv, [dv])
            a = jnp.where(a > 0, a, a * 0.2)
            w = jnp.exp(a)
            wv[pl.ds(j * 16, 16)] = w
            plsc.addupdate_scatter(denv, [dv], w)
            return 0
        lax.fori_loop(0, EPW // 16, edge16, 0)

        for t in range((EPWP - EPW) // 16):
            wv[pl.ds(EPW + t * 16, 16)] = zeros16

        pltpu.sync_copy(wv, w_out.at[wid, 0])
        pltpu.sync_copy(denv, den_out.at[wid, 0])

    w3, den = k(a_src, a_dst, srcf, dstf)
    return w3, den.reshape(NW, N_NODES)


def _agg_sc(h, w4, srcf, dst4):
    @functools.partial(
        pl.kernel,
        out_type=jax.ShapeDtypeStruct((NC, N_NODES, D), jnp.float32),
        mesh=plsc.VectorSubcoreMesh(**_SC_MESH),
        scratch_types=dict(
            src1=pltpu.VMEM((EPWP,), jnp.int32),
            db2=pltpu.VMEM((2, CBH), jnp.int32),
            wb2=pltpu.VMEM((2, CBH), jnp.float32),
            r0=pltpu.VMEM((CBH, D // 2), jnp.float32),
            r1=pltpu.VMEM((CBH, D // 2), jnp.float32),
            ro0=pltpu.VMEM((CBH, D), jnp.float32),
            ro1=pltpu.VMEM((CBH, D), jnp.float32),
            acc_s=pltpu.VMEM_SHARED((N_NODES, D), jnp.float32),
            sdw=pltpu.SemaphoreType.DMA,
            sg0=pltpu.SemaphoreType.DMA,
            sg1=pltpu.SemaphoreType.DMA,
            ss0=pltpu.SemaphoreType.DMA,
            ss1=pltpu.SemaphoreType.DMA,
        ),
        compiler_params=pltpu.CompilerParams(needs_layout_passes=False, use_tc_tiling_on_sc=False),
    )
    def k(h_hbm, w_hbm, src_hbm, dst_hbm, acc_out,
          src1, db2, wb2, r0, r1, ro0, ro1, acc_s, sdw, sg0, sg1, ss0, ss1):
        cid = lax.axis_index("c")
        sid = lax.axis_index("s")
        wid = sid * NC + cid

        pltpu.sync_copy(src_hbm.at[wid, 0], src1)

        zeros16 = jnp.zeros((16,), jnp.float32)

        for i in range(16):
            for t in range(D // 16):
                ro0[i, pl.ds(t * 16, 16)] = zeros16

        def zacc(i, _):
            pltpu.sync_copy(ro0.at[pl.ds(0, 16)],
                            acc_s.at[pl.ds(sid * RPS + i * 16, 16)])
            return 0
        lax.fori_loop(0, RPS // 16, zacc, 0)

        @pl.when(sid == NS - 1)
        def _():
            pltpu.sync_copy(ro0.at[pl.ds(0, 16)], acc_s.at[pl.ds(NS * RPS, TAIL)])

        plsc.subcore_barrier()

        def scale(rp, ro, half):
            for g in range(CBH // 16):
                wv16 = wb2[half, pl.ds(g * 16, 16)]
                for r in range(16):
                    ws = jnp.full((16,), wv16[r], jnp.float32)
                    row = g * 16 + r
                    for t in range(D // 32):
                        v = rp[row, pl.ds(t * 16, 16)]
                        a, b = plsc.unpack(plsc.bitcast(v, jnp.bfloat16),
                                           format=plsc.PackFormat.INTERLEAVED)
                        ro[row, pl.ds(t * 32, 16)] = a * ws
                        ro[row, pl.ds(t * 32 + 16, 16)] = b * ws

        def wait_g(rbuf, sem):
            pltpu.make_async_copy(h_hbm.at[pl.ds(0, CBH)], rbuf, sem).wait()

        pltpu.async_copy(dst_hbm.at[wid, 0], db2, sdw)
        pltpu.async_copy(w_hbm.at[wid, 0], wb2, sdw)
        pltpu.async_copy(h_hbm.at[src1.at[pl.ds(0, CBH)]], r0, sg0)
        pltpu.async_copy(h_hbm.at[src1.at[pl.ds(CBH, CBH)]], r1, sg1)

        def pair_body(t, _):
            base = t * 2 * CBH
            wait_g(r0, sg0)
            pltpu.make_async_copy(dst_hbm.at[wid, 0], db2, sdw).wait()
            pltpu.make_async_copy(w_hbm.at[wid, 0], wb2, sdw).wait()
            scale(r0, ro0, 0)
            s0 = pltpu.async_copy(ro0, acc_s.at[db2.at[0]], ss0, add=True)
            wait_g(r1, sg1)
            scale(r1, ro1, 1)
            s1 = pltpu.async_copy(ro1, acc_s.at[db2.at[1]], ss1, add=True)

            @pl.when(t < PAIRS - 1)
            def _():
                pltpu.async_copy(
                    h_hbm.at[src1.at[pl.ds(base + 2 * CBH, CBH)]], r0, sg0)
                pltpu.async_copy(
                    h_hbm.at[src1.at[pl.ds(base + 3 * CBH, CBH)]], r1, sg1)

            s0.wait()
            s1.wait()

            @pl.when(t < PAIRS - 1)
            def _():
                pltpu.async_copy(dst_hbm.at[wid, t + 1], db2, sdw)
                pltpu.async_copy(w_hbm.at[wid, t + 1], wb2, sdw)
            return 0
        lax.fori_loop(0, PAIRS, pair_body, 0)

        plsc.subcore_barrier()
        pltpu.sync_copy(acc_s.at[pl.ds(sid * RPS, RPS)],
                        acc_out.at[cid].at[pl.ds(sid * RPS, RPS)])

        @pl.when(sid == NS - 1)
        def _():
            pltpu.sync_copy(acc_s.at[pl.ds(NS * RPS, TAIL)],
                            acc_out.at[cid].at[pl.ds(NS * RPS, TAIL)])

    return k(h, w4, srcf, dst4)


def _pack_h(h):
    n = h.shape[0]
    hb = h.astype(jnp.bfloat16).reshape(n, D // 32, 2, 16)
    st = jnp.stack([hb[:, :, 0, :], hb[:, :, 1, :]], axis=-1)
    return lax.bitcast_convert_type(st, jnp.float32).reshape(n, D // 2)


def _edge_sc(h, a_src, a_dst, srcf, dstf, dst4):
    w3, den32 = _attn_sc(a_src, a_dst, srcf, dstf)
    w4 = w3.reshape(NW, PAIRS, 2, CBH)
    acc2 = _agg_sc(_pack_h(h), w4, srcf, dst4)
    return acc2, den32


def _augment(W, att_src, att_dst):
    A = jnp.zeros((D, D), jnp.float32)
    A = A.at[:, 0].set(att_src).at[:, 1].set(att_dst)
    return jnp.concatenate([W, W @ A], axis=1)


def kernel(x, edge_index, W1, att_src1, att_dst1, b1, W2, att_src2, att_dst2, b2):
    pad = ((0, 0), (0, EPWP - EPW))
    srcf = jnp.pad(edge_index[0].reshape(NW, EPW), pad).reshape(NW, 1, EPWP)
    dstf = jnp.pad(edge_index[1].reshape(NW, EPW), pad).reshape(NW, 1, EPWP)
    dst4 = dstf.reshape(NW, PAIRS, 2, CBH)

    wc1 = _augment(W1, att_src1, att_dst1)
    out1 = _mm(x, wc1)
    h1 = out1[:, :D]
    as1 = out1[:, D]
    ad1 = out1[:, D + 1]
    acc1, den1 = _edge_sc(h1, as1, ad1, srcf, dstf, dst4)

    wc2 = _augment(W2, att_src2, att_dst2)
    out2 = _norm_mm(acc1, den1, b1, wc2)
    h2 = out2[:, :D]
    as2 = out2[:, D]
    ad2 = out2[:, D + 1]
    acc2, den2 = _edge_sc(h2, as2, ad2, srcf, dstf, dst4)

    return _norm(acc2, den2, b2)


# speedup vs baseline: 1.4370x; 1.0027x over previous
import functools

import jax
import jax.numpy as jnp
from jax import lax
from jax.experimental import pallas as pl
from jax.experimental.pallas import tpu as pltpu
from jax.experimental.pallas import tpu_sc as plsc

N_NODES = 10000
D = 128
BM = 1000

NC = 2
NS = 16
NW = NC * NS
E = 320000
EPW = E // NW
CBH = 64
EPWP = 10112
PAIRS = EPWP // (2 * CBH)
RPS = 624
TAIL = N_NODES - NS * RPS


def _mm_body(x_ref, w_ref, a_ref, h_ref, aux_ref):
    h = jnp.dot(x_ref[...], w_ref[...], preferred_element_type=jnp.float32)
    h_ref[...] = h
    aux_ref[...] = jnp.dot(h, a_ref[...], preferred_element_type=jnp.float32)


def _mm(x, w, a2):
    m, k = x.shape
    return pl.pallas_call(
        _mm_body,
        grid=(m // BM,),
        in_specs=[
            pl.BlockSpec((BM, k), lambda i: (i, 0)),
            pl.BlockSpec((k, D), lambda i: (0, 0)),
            pl.BlockSpec((D, D), lambda i: (0, 0)),
        ],
        out_specs=[
            pl.BlockSpec((BM, D), lambda i: (i, 0)),
            pl.BlockSpec((BM, D), lambda i: (i, 0)),
        ],
        out_shape=[
            jax.ShapeDtypeStruct((m, D), jnp.float32),
            jax.ShapeDtypeStruct((m, D), jnp.float32),
        ],
    )(x, w, a2)


def _norm_mm_body(a_ref, d_ref, b_ref, w_ref, a2_ref, h_ref, aux_ref):
    den = jnp.sum(d_ref[...], axis=1) + 1e-16
    g = (a_ref[0] + a_ref[1]) / den[:, None] + b_ref[...]
    g = jnp.maximum(g, 0.0)
    h = jnp.dot(g, w_ref[...], preferred_element_type=jnp.float32)
    h_ref[...] = h
    aux_ref[...] = jnp.dot(h, a2_ref[...], preferred_element_type=jnp.float32)


def _norm_mm(acc2, den32, b, w, a2):
    m = acc2.shape[1]
    return pl.pallas_call(
        _norm_mm_body,
        grid=(m // BM,),
        in_specs=[
            pl.BlockSpec((2, BM, D), lambda i: (0, i, 0)),
            pl.BlockSpec((BM, NW), lambda i: (i, 0)),
            pl.BlockSpec((1, D), lambda i: (0, 0)),
            pl.BlockSpec((D, D), lambda i: (0, 0)),
            pl.BlockSpec((D, D), lambda i: (0, 0)),
        ],
        out_specs=[
            pl.BlockSpec((BM, D), lambda i: (i, 0)),
            pl.BlockSpec((BM, D), lambda i: (i, 0)),
        ],
        out_shape=[
            jax.ShapeDtypeStruct((m, D), jnp.float32),
            jax.ShapeDtypeStruct((m, D), jnp.float32),
        ],
    )(acc2, den32.T, b.reshape(1, D), w, a2)


def _norm_body(a_ref, d_ref, b_ref, o_ref):
    den = jnp.sum(d_ref[...], axis=1) + 1e-16
    o_ref[...] = (a_ref[0] + a_ref[1]) / den[:, None] + b_ref[...]


def _norm(acc2, den32, b):
    m = acc2.shape[1]
    return pl.pallas_call(
        _norm_body,
        grid=(m // BM,),
        in_specs=[
            pl.BlockSpec((2, BM, D), lambda i: (0, i, 0)),
            pl.BlockSpec((BM, NW), lambda i: (i, 0)),
            pl.BlockSpec((1, D), lambda i: (0, 0)),
        ],
        out_specs=pl.BlockSpec((BM, D), lambda i: (i, 0)),
        out_shape=jax.ShapeDtypeStruct((m, D), jnp.float32),
    )(acc2, den32.T, b.reshape(1, D))


_SC_MESH = dict(core_axis_name="c", subcore_axis_name="s", num_cores=NC,
                num_subcores=NS)


def _attn_sc(a_src, a_dst, srcf, dstf):
    @functools.partial(
        pl.kernel,
        out_type=(
            jax.ShapeDtypeStruct((NW, 1, EPWP), jnp.float32),
            jax.ShapeDtypeStruct((NW, 1, N_NODES), jnp.float32),
        ),
        mesh=plsc.VectorSubcoreMesh(**_SC_MESH),
        scratch_types=dict(
            src1=pltpu.VMEM((EPWP,), jnp.int32),
            dst1=pltpu.VMEM((EPWP,), jnp.int32),
            asv=pltpu.VMEM((N_NODES,), jnp.float32),
            adv=pltpu.VMEM((N_NODES,), jnp.float32),
            wv=pltpu.VMEM((EPWP,), jnp.float32),
            denv=pltpu.VMEM((N_NODES,), jnp.float32),
        ),
        compiler_params=pltpu.CompilerParams(needs_layout_passes=False),
    )
    def k(asrc_hbm, adst_hbm, src_hbm, dst_hbm, w_out, den_out,
          src1, dst1, asv, adv, wv, denv):
        cid = lax.axis_index("c")
        sid = lax.axis_index("s")
        wid = sid * NC + cid

        pltpu.sync_copy(src_hbm.at[wid, 0], src1)
        pltpu.sync_copy(dst_hbm.at[wid, 0], dst1)
        pltpu.sync_copy(asrc_hbm, asv)
        pltpu.sync_copy(adst_hbm, adv)

        zeros16 = jnp.zeros((16,), jnp.float32)

        def zden(i, _):
            for t in range(5):
                denv[pl.ds(i * 80 + t * 16, 16)] = zeros16
            return 0
        lax.fori_loop(0, N_NODES // 80, zden, 0)

        def edge16(e0):
            sv = src1[pl.ds(e0, 16)]
            dv = dst1[pl.ds(e0, 16)]
            a = plsc.load_gather(asv, [sv]) + plsc.load_gather(adv, [dv])
            a = jnp.where(a > 0, a, a * 0.2)
            w = jnp.exp(a)
            wv[pl.ds(e0, 16)] = w
            plsc.addupdate_scatter(denv, [dv], w)

        def edge32(j, _):
            edge16(j * 32)
            edge16(j * 32 + 16)
            return 0
        lax.fori_loop(0, EPW // 32, edge32, 0)
        edge16(EPW - 16)

        for t in range((EPWP - EPW) // 16):
            wv[pl.ds(EPW + t * 16, 16)] = zeros16

        pltpu.sync_copy(wv, w_out.at[wid, 0])
        pltpu.sync_copy(denv, den_out.at[wid, 0])

    w3, den = k(a_src, a_dst, srcf, dstf)
    return w3, den.reshape(NW, N_NODES)


def _agg_sc(h, w4, srcf, dst4):
    @functools.partial(
        pl.kernel,
        out_type=jax.ShapeDtypeStruct((NC, N_NODES, D), jnp.float32),
        mesh=plsc.VectorSubcoreMesh(**_SC_MESH),
        scratch_types=dict(
            src1=pltpu.VMEM((EPWP,), jnp.int32),
            db2=pltpu.VMEM((2, CBH), jnp.int32),
            wb2=pltpu.VMEM((2, CBH), jnp.float32),
            r0=pltpu.VMEM((CBH, D // 2), jnp.float32),
            r1=pltpu.VMEM((CBH, D // 2), jnp.float32),
            ro0=pltpu.VMEM((CBH, D), jnp.float32),
            ro1=pltpu.VMEM((CBH, D), jnp.float32),
            acc_s=pltpu.VMEM_SHARED((N_NODES, D), jnp.float32),
            sdw=pltpu.SemaphoreType.DMA,
            sg0=pltpu.SemaphoreType.DMA,
            sg1=pltpu.SemaphoreType.DMA,
            ss0=pltpu.SemaphoreType.DMA,
            ss1=pltpu.SemaphoreType.DMA,
        ),
        compiler_params=pltpu.CompilerParams(needs_layout_passes=False, use_tc_tiling_on_sc=False),
    )
    def k(h_hbm, w_hbm, src_hbm, dst_hbm, acc_out,
          src1, db2, wb2, r0, r1, ro0, ro1, acc_s, sdw, sg0, sg1, ss0, ss1):
        cid = lax.axis_index("c")
        sid = lax.axis_index("s")
        wid = sid * NC + cid

        pltpu.sync_copy(src_hbm.at[wid, 0], src1)

        zeros16 = jnp.zeros((16,), jnp.float32)

        for i in range(16):
            for t in range(D // 16):
                ro0[i, pl.ds(t * 16, 16)] = zeros16

        def zacc(i, _):
            pltpu.sync_copy(ro0.at[pl.ds(0, 16)],
                            acc_s.at[pl.ds(sid * RPS + i * 16, 16)])
            return 0
        lax.fori_loop(0, RPS // 16, zacc, 0)

        @pl.when(sid == NS - 1)
        def _():
            pltpu.sync_copy(ro0.at[pl.ds(0, 16)], acc_s.at[pl.ds(NS * RPS, TAIL)])

        plsc.subcore_barrier()

        def scale(rp, ro, half):
            for g in range(CBH // 16):
                wv16 = wb2[half, pl.ds(g * 16, 16)]
                for r in range(16):
                    ws = jnp.full((16,), wv16[r], jnp.float32)
                    row = g * 16 + r
                    for t in range(D // 32):
                        v = rp[row, pl.ds(t * 16, 16)]
                        a, b = plsc.unpack(plsc.bitcast(v, jnp.bfloat16),
                                           format=plsc.PackFormat.INTERLEAVED)
                        ro[row, pl.ds(t * 32, 16)] = a * ws
                        ro[row, pl.ds(t * 32 + 16, 16)] = b * ws

        def wait_g(rbuf, sem):
            pltpu.make_async_copy(h_hbm.at[pl.ds(0, CBH)], rbuf, sem).wait()

        pltpu.async_copy(dst_hbm.at[wid, 0], db2, sdw)
        pltpu.async_copy(w_hbm.at[wid, 0], wb2, sdw)
        pltpu.async_copy(h_hbm.at[src1.at[pl.ds(0, CBH)]], r0, sg0)
        pltpu.async_copy(h_hbm.at[src1.at[pl.ds(CBH, CBH)]], r1, sg1)

        def pair_body(t, _):
            base = t * 2 * CBH
            wait_g(r0, sg0)
            pltpu.make_async_copy(dst_hbm.at[wid, 0], db2, sdw).wait()
            pltpu.make_async_copy(w_hbm.at[wid, 0], wb2, sdw).wait()
            scale(r0, ro0, 0)
            s0 = pltpu.async_copy(ro0, acc_s.at[db2.at[0]], ss0, add=True)
            wait_g(r1, sg1)
            scale(r1, ro1, 1)
            s1 = pltpu.async_copy(ro1, acc_s.at[db2.at[1]], ss1, add=True)

            @pl.when(t < PAIRS - 1)
            def _():
                pltpu.async_copy(
                    h_hbm.at[src1.at[pl.ds(base + 2 * CBH, CBH)]], r0, sg0)
                pltpu.async_copy(
                    h_hbm.at[src1.at[pl.ds(base + 3 * CBH, CBH)]], r1, sg1)

            s0.wait()
            s1.wait()

            @pl.when(t < PAIRS - 1)
            def _():
                pltpu.async_copy(dst_hbm.at[wid, t + 1], db2, sdw)
                pltpu.async_copy(w_hbm.at[wid, t + 1], wb2, sdw)
            return 0
        lax.fori_loop(0, PAIRS, pair_body, 0)

        plsc.subcore_barrier()
        pltpu.sync_copy(acc_s.at[pl.ds(sid * RPS, RPS)],
                        acc_out.at[cid].at[pl.ds(sid * RPS, RPS)])

        @pl.when(sid == NS - 1)
        def _():
            pltpu.sync_copy(acc_s.at[pl.ds(NS * RPS, TAIL)],
                            acc_out.at[cid].at[pl.ds(NS * RPS, TAIL)])

    return k(h, w4, srcf, dst4)


def _pack_h(h):
    n = h.shape[0]
    hb = h.astype(jnp.bfloat16).reshape(n, D // 32, 2, 16)
    st = jnp.stack([hb[:, :, 0, :], hb[:, :, 1, :]], axis=-1)
    return lax.bitcast_convert_type(st, jnp.float32).reshape(n, D // 2)


def _edge_sc(h, a_src, a_dst, srcf, dstf, dst4):
    w3, den32 = _attn_sc(a_src, a_dst, srcf, dstf)
    w4 = w3.reshape(NW, PAIRS, 2, CBH)
    acc2 = _agg_sc(_pack_h(h), w4, srcf, dst4)
    return acc2, den32


def _attmat(att_src, att_dst):
    A = jnp.zeros((D, D), jnp.float32)
    return A.at[:, 0].set(att_src).at[:, 1].set(att_dst)


def kernel(x, edge_index, W1, att_src1, att_dst1, b1, W2, att_src2, att_dst2, b2):
    pad = ((0, 0), (0, EPWP - EPW))
    srcf = jnp.pad(edge_index[0].reshape(NW, EPW), pad).reshape(NW, 1, EPWP)
    dstf = jnp.pad(edge_index[1].reshape(NW, EPW), pad).reshape(NW, 1, EPWP)
    dst4 = dstf.reshape(NW, PAIRS, 2, CBH)

    h1, aux1 = _mm(x, W1, _attmat(att_src1, att_dst1))
    acc1, den1 = _edge_sc(h1, aux1[:, 0], aux1[:, 1], srcf, dstf, dst4)

    h2, aux2 = _norm_mm(acc1, den1, b1, W2, _attmat(att_src2, att_dst2))
    acc2, den2 = _edge_sc(h2, aux2[:, 0], aux2[:, 1], srcf, dstf, dst4)

    return _norm(acc2, den2, b2)


# speedup vs baseline: 1.4528x; 1.0110x over previous
import functools

import jax
import jax.numpy as jnp
from jax import lax
from jax.experimental import pallas as pl
from jax.experimental.pallas import tpu as pltpu
from jax.experimental.pallas import tpu_sc as plsc

N_NODES = 10000
D = 128
BM = 1000

NC = 2
NS = 16
NW = NC * NS
E = 320000
EPW = E // NW
CBH = 64
EPWP = 10112
PAIRS = EPWP // (2 * CBH)
RPS = 624
TAIL = N_NODES - NS * RPS


def _mm_body(x_ref, w_ref, a_ref, h_ref, aux_ref):
    h = jnp.dot(x_ref[...], w_ref[...], preferred_element_type=jnp.float32)
    h_ref[...] = h
    aux_ref[...] = jnp.dot(h, a_ref[...], preferred_element_type=jnp.float32)


def _mm(x, w, a2):
    m, k = x.shape
    return pl.pallas_call(
        _mm_body,
        grid=(m // BM,),
        in_specs=[
            pl.BlockSpec((BM, k), lambda i: (i, 0)),
            pl.BlockSpec((k, D), lambda i: (0, 0)),
            pl.BlockSpec((D, D), lambda i: (0, 0)),
        ],
        out_specs=[
            pl.BlockSpec((BM, D), lambda i: (i, 0)),
            pl.BlockSpec((BM, D), lambda i: (i, 0)),
        ],
        out_shape=[
            jax.ShapeDtypeStruct((m, D), jnp.float32),
            jax.ShapeDtypeStruct((m, D), jnp.float32),
        ],
    )(x, w, a2)


def _norm_mm_body(a_ref, d_ref, b_ref, w_ref, a2_ref, h_ref, aux_ref):
    den = jnp.sum(d_ref[...], axis=1) + 1e-16
    g = (a_ref[0] + a_ref[1]) / den[:, None] + b_ref[...]
    g = jnp.maximum(g, 0.0)
    h = jnp.dot(g, w_ref[...], preferred_element_type=jnp.float32)
    h_ref[...] = h
    aux_ref[...] = jnp.dot(h, a2_ref[...], preferred_element_type=jnp.float32)


def _norm_mm(acc2, den32, b, w, a2):
    m = acc2.shape[1]
    return pl.pallas_call(
        _norm_mm_body,
        grid=(m // BM,),
        in_specs=[
            pl.BlockSpec((2, BM, D), lambda i: (0, i, 0)),
            pl.BlockSpec((BM, NW), lambda i: (i, 0)),
            pl.BlockSpec((1, D), lambda i: (0, 0)),
            pl.BlockSpec((D, D), lambda i: (0, 0)),
            pl.BlockSpec((D, D), lambda i: (0, 0)),
        ],
        out_specs=[
            pl.BlockSpec((BM, D), lambda i: (i, 0)),
            pl.BlockSpec((BM, D), lambda i: (i, 0)),
        ],
        out_shape=[
            jax.ShapeDtypeStruct((m, D), jnp.float32),
            jax.ShapeDtypeStruct((m, D), jnp.float32),
        ],
    )(acc2, den32.T, b.reshape(1, D), w, a2)


def _norm_body(a_ref, d_ref, b_ref, o_ref):
    den = jnp.sum(d_ref[...], axis=1) + 1e-16
    o_ref[...] = (a_ref[0] + a_ref[1]) / den[:, None] + b_ref[...]


def _norm(acc2, den32, b):
    m = acc2.shape[1]
    return pl.pallas_call(
        _norm_body,
        grid=(m // BM,),
        in_specs=[
            pl.BlockSpec((2, BM, D), lambda i: (0, i, 0)),
            pl.BlockSpec((BM, NW), lambda i: (i, 0)),
            pl.BlockSpec((1, D), lambda i: (0, 0)),
        ],
        out_specs=pl.BlockSpec((BM, D), lambda i: (i, 0)),
        out_shape=jax.ShapeDtypeStruct((m, D), jnp.float32),
    )(acc2, den32.T, b.reshape(1, D))


_SC_MESH = dict(core_axis_name="c", subcore_axis_name="s", num_cores=NC,
                num_subcores=NS)


def _attn_sc(a_src, a_dst, srcf, dstf):
    @functools.partial(
        pl.kernel,
        out_type=(
            jax.ShapeDtypeStruct((NW, 1, EPWP), jnp.float32),
            jax.ShapeDtypeStruct((NW, 1, N_NODES), jnp.float32),
        ),
        mesh=plsc.VectorSubcoreMesh(**_SC_MESH),
        scratch_types=dict(
            src1=pltpu.VMEM((EPWP,), jnp.int32),
            dst1=pltpu.VMEM((EPWP,), jnp.int32),
            asv=pltpu.VMEM((N_NODES,), jnp.float32),
            adv=pltpu.VMEM((N_NODES,), jnp.float32),
            wv=pltpu.VMEM((EPWP,), jnp.float32),
            denv=pltpu.VMEM((N_NODES,), jnp.float32),
        ),
        compiler_params=pltpu.CompilerParams(needs_layout_passes=False),
    )
    def k(asrc_hbm, adst_hbm, src_hbm, dst_hbm, w_out, den_out,
          src1, dst1, asv, adv, wv, denv):
        cid = lax.axis_index("c")
        sid = lax.axis_index("s")
        wid = sid * NC + cid

        pltpu.sync_copy(src_hbm.at[wid, 0], src1)
        pltpu.sync_copy(dst_hbm.at[wid, 0], dst1)
        pltpu.sync_copy(asrc_hbm, asv)
        pltpu.sync_copy(adst_hbm, adv)

        zeros16 = jnp.zeros((16,), jnp.float32)

        def zden(i, _):
            for t in range(5):
                denv[pl.ds(i * 80 + t * 16, 16)] = zeros16
            return 0
        lax.fori_loop(0, N_NODES // 80, zden, 0)

        def edge16(e0):
            sv = src1[pl.ds(e0, 16)]
            dv = dst1[pl.ds(e0, 16)]
            a = plsc.load_gather(asv, [sv]) + plsc.load_gather(adv, [dv])
            a = jnp.where(a > 0, a, a * 0.2)
            w = jnp.exp(a)
            wv[pl.ds(e0, 16)] = w
            plsc.addupdate_scatter(denv, [dv], w)

        def edge32(j, _):
            edge16(j * 32)
            edge16(j * 32 + 16)
            return 0
        lax.fori_loop(0, EPW // 32, edge32, 0)
        edge16(EPW - 16)

        for t in range((EPWP - EPW) // 16):
            wv[pl.ds(EPW + t * 16, 16)] = zeros16

        pltpu.sync_copy(wv, w_out.at[wid, 0])
        pltpu.sync_copy(denv, den_out.at[wid, 0])

    w3, den = k(a_src, a_dst, srcf, dstf)
    return w3, den.reshape(NW, N_NODES)


def _agg_sc(h, w4, srcf, dst4):
    @functools.partial(
        pl.kernel,
        out_type=jax.ShapeDtypeStruct((NC, N_NODES, D), jnp.float32),
        mesh=plsc.VectorSubcoreMesh(**_SC_MESH),
        scratch_types=dict(
            src1=pltpu.VMEM((EPWP,), jnp.int32),
            db2=pltpu.VMEM((2, CBH), jnp.int32),
            wb2=pltpu.VMEM((2, CBH), jnp.float32),
            r0=pltpu.VMEM((CBH, D // 2), jnp.float32),
            r1=pltpu.VMEM((CBH, D // 2), jnp.float32),
            ro0=pltpu.VMEM((CBH, D), jnp.float32),
            ro1=pltpu.VMEM((CBH, D), jnp.float32),
            acc_s=pltpu.VMEM_SHARED((N_NODES, D), jnp.float32),
            sdw=pltpu.SemaphoreType.DMA,
            sg0=pltpu.SemaphoreType.DMA,
            sg1=pltpu.SemaphoreType.DMA,
            ss0=pltpu.SemaphoreType.DMA,
            ss1=pltpu.SemaphoreType.DMA,
        ),
        compiler_params=pltpu.CompilerParams(needs_layout_passes=False, use_tc_tiling_on_sc=False),
    )
    def k(h_hbm, w_hbm, src_hbm, dst_hbm, acc_out,
          src1, db2, wb2, r0, r1, ro0, ro1, acc_s, sdw, sg0, sg1, ss0, ss1):
        cid = lax.axis_index("c")
        sid = lax.axis_index("s")
        wid = sid * NC + cid

        pltpu.sync_copy(src_hbm.at[wid, 0], src1)

        zeros16 = jnp.zeros((16,), jnp.float32)

        for i in range(16):
            for t in range(D // 16):
                ro0[i, pl.ds(t * 16, 16)] = zeros16

        def zacc(i, _):
            pltpu.sync_copy(ro0.at[pl.ds(0, 16)],
                            acc_s.at[pl.ds(sid * RPS + i * 16, 16)])
            return 0
        lax.fori_loop(0, RPS // 16, zacc, 0)

        @pl.when(sid == NS - 1)
        def _():
            pltpu.sync_copy(ro0.at[pl.ds(0, 16)], acc_s.at[pl.ds(NS * RPS, TAIL)])

        plsc.subcore_barrier()

        def scale(rp, ro, half):
            for g in range(CBH // 16):
                wv16 = wb2[half, pl.ds(g * 16, 16)]
                for r in range(16):
                    ws = jnp.full((16,), wv16[r], jnp.float32)
                    row = g * 16 + r
                    for t in range(D // 32):
                        v = rp[row, pl.ds(t * 16, 16)]
                        a, b = plsc.unpack(plsc.bitcast(v, jnp.bfloat16),
                                           format=plsc.PackFormat.INTERLEAVED)
                        ro[row, pl.ds(t * 32, 16)] = a * ws
                        ro[row, pl.ds(t * 32 + 16, 16)] = b * ws

        def wait_g(rbuf, sem):
            pltpu.make_async_copy(h_hbm.at[pl.ds(0, CBH)], rbuf, sem).wait()

        pltpu.async_copy(dst_hbm.at[wid, 0], db2, sdw)
        pltpu.async_copy(w_hbm.at[wid, 0], wb2, sdw)
        pltpu.async_copy(h_hbm.at[src1.at[pl.ds(0, CBH)]], r0, sg0)
        pltpu.async_copy(h_hbm.at[src1.at[pl.ds(CBH, CBH)]], r1, sg1)

        def pair_body(t, _):
            base = t * 2 * CBH
            wait_g(r0, sg0)
            pltpu.make_async_copy(dst_hbm.at[wid, 0], db2, sdw).wait()
            pltpu.make_async_copy(w_hbm.at[wid, 0], wb2, sdw).wait()
            scale(r0, ro0, 0)
            s0 = pltpu.async_copy(ro0, acc_s.at[db2.at[0]], ss0, add=True)

            @pl.when(t < PAIRS - 1)
            def _():
                pltpu.async_copy(
                    h_hbm.at[src1.at[pl.ds(base + 2 * CBH, CBH)]], r0, sg0)

            wait_g(r1, sg1)
            scale(r1, ro1, 1)
            s1 = pltpu.async_copy(ro1, acc_s.at[db2.at[1]], ss1, add=True)

            @pl.when(t < PAIRS - 1)
            def _():
                pltpu.async_copy(
                    h_hbm.at[src1.at[pl.ds(base + 3 * CBH, CBH)]], r1, sg1)

            s0.wait()
            s1.wait()

            @pl.when(t < PAIRS - 1)
            def _():
                pltpu.async_copy(dst_hbm.at[wid, t + 1], db2, sdw)
                pltpu.async_copy(w_hbm.at[wid, t + 1], wb2, sdw)
            return 0
        lax.fori_loop(0, PAIRS, pair_body, 0)

        plsc.subcore_barrier()
        pltpu.sync_copy(acc_s.at[pl.ds(sid * RPS, RPS)],
                        acc_out.at[cid].at[pl.ds(sid * RPS, RPS)])

        @pl.when(sid == NS - 1)
        def _():
            pltpu.sync_copy(acc_s.at[pl.ds(NS * RPS, TAIL)],
                            acc_out.at[cid].at[pl.ds(NS * RPS, TAIL)])

    return k(h, w4, srcf, dst4)


def _pack_h(h):
    n = h.shape[0]
    hb = h.astype(jnp.bfloat16).reshape(n, D // 32, 2, 16)
    st = jnp.stack([hb[:, :, 0, :], hb[:, :, 1, :]], axis=-1)
    return lax.bitcast_convert_type(st, jnp.float32).reshape(n, D // 2)


def _edge_sc(h, a_src, a_dst, srcf, dstf, dst4):
    w3, den32 = _attn_sc(a_src, a_dst, srcf, dstf)
    w4 = w3.reshape(NW, PAIRS, 2, CBH)
    acc2 = _agg_sc(_pack_h(h), w4, srcf, dst4)
    return acc2, den32


def _attmat(att_src, att_dst):
    A = jnp.zeros((D, D), jnp.float32)
    return A.at[:, 0].set(att_src).at[:, 1].set(att_dst)


def kernel(x, edge_index, W1, att_src1, att_dst1, b1, W2, att_src2, att_dst2, b2):
    pad = ((0, 0), (0, EPWP - EPW))
    srcf = jnp.pad(edge_index[0].reshape(NW, EPW), pad).reshape(NW, 1, EPWP)
    dstf = jnp.pad(edge_index[1].reshape(NW, EPW), pad).reshape(NW, 1, EPWP)
    dst4 = dstf.reshape(NW, PAIRS, 2, CBH)

    h1, aux1 = _mm(x, W1, _attmat(att_src1, att_dst1))
    acc1, den1 = _edge_sc(h1, aux1[:, 0], aux1[:, 1], srcf, dstf, dst4)

    h2, aux2 = _norm_mm(acc1, den1, b1, W2, _attmat(att_src2, att_dst2))
    acc2, den2 = _edge_sc(h2, aux2[:, 0], aux2[:, 1], srcf, dstf, dst4)

    return _norm(acc2, den2, b2)


# speedup vs baseline: 1.4544x; 1.0011x over previous
import functools

import jax
import jax.numpy as jnp
from jax import lax
from jax.experimental import pallas as pl
from jax.experimental.pallas import tpu as pltpu
from jax.experimental.pallas import tpu_sc as plsc

N_NODES = 10000
D = 128
BM = 1000

NC = 2
NS = 16
NW = NC * NS
E = 320000
EPW = E // NW
CBH = 64
EPWP = 10112
PAIRS = EPWP // (2 * CBH)
RPS = 624
TAIL = N_NODES - NS * RPS


def _mm_body(x_ref, w_ref, a_ref, h_ref, aux_ref):
    h = jnp.dot(x_ref[...], w_ref[...], preferred_element_type=jnp.float32)
    h_ref[...] = h
    aux_ref[...] = jnp.dot(h, a_ref[...], preferred_element_type=jnp.float32)


def _mm(x, w, a2):
    m, k = x.shape
    return pl.pallas_call(
        _mm_body,
        grid=(m // BM,),
        in_specs=[
            pl.BlockSpec((BM, k), lambda i: (i, 0)),
            pl.BlockSpec((k, D), lambda i: (0, 0)),
            pl.BlockSpec((D, D), lambda i: (0, 0)),
        ],
        out_specs=[
            pl.BlockSpec((BM, D), lambda i: (i, 0)),
            pl.BlockSpec((BM, D), lambda i: (i, 0)),
        ],
        out_shape=[
            jax.ShapeDtypeStruct((m, D), jnp.float32),
            jax.ShapeDtypeStruct((m, D), jnp.float32),
        ],
    )(x, w, a2)


def _norm_mm_body(a_ref, d_ref, b_ref, w_ref, a2_ref, h_ref, aux_ref):
    den = jnp.sum(d_ref[...], axis=1) + 1e-16
    g = (a_ref[0] + a_ref[1]) / den[:, None] + b_ref[...]
    g = jnp.maximum(g, 0.0)
    h = jnp.dot(g, w_ref[...], preferred_element_type=jnp.float32)
    h_ref[...] = h
    aux_ref[...] = jnp.dot(h, a2_ref[...], preferred_element_type=jnp.float32)


def _norm_mm(acc2, den32, b, w, a2):
    m = acc2.shape[1]
    return pl.pallas_call(
        _norm_mm_body,
        grid=(m // BM,),
        in_specs=[
            pl.BlockSpec((2, BM, D), lambda i: (0, i, 0)),
            pl.BlockSpec((BM, NW), lambda i: (i, 0)),
            pl.BlockSpec((1, D), lambda i: (0, 0)),
            pl.BlockSpec((D, D), lambda i: (0, 0)),
            pl.BlockSpec((D, D), lambda i: (0, 0)),
        ],
        out_specs=[
            pl.BlockSpec((BM, D), lambda i: (i, 0)),
            pl.BlockSpec((BM, D), lambda i: (i, 0)),
        ],
        out_shape=[
            jax.ShapeDtypeStruct((m, D), jnp.float32),
            jax.ShapeDtypeStruct((m, D), jnp.float32),
        ],
    )(acc2, den32.T, b.reshape(1, D), w, a2)


def _norm_body(a_ref, d_ref, b_ref, o_ref):
    den = jnp.sum(d_ref[...], axis=1) + 1e-16
    o_ref[...] = (a_ref[0] + a_ref[1]) / den[:, None] + b_ref[...]


def _norm(acc2, den32, b):
    m = acc2.shape[1]
    return pl.pallas_call(
        _norm_body,
        grid=(m // BM,),
        in_specs=[
            pl.BlockSpec((2, BM, D), lambda i: (0, i, 0)),
            pl.BlockSpec((BM, NW), lambda i: (i, 0)),
            pl.BlockSpec((1, D), lambda i: (0, 0)),
        ],
        out_specs=pl.BlockSpec((BM, D), lambda i: (i, 0)),
        out_shape=jax.ShapeDtypeStruct((m, D), jnp.float32),
    )(acc2, den32.T, b.reshape(1, D))


_SC_MESH = dict(core_axis_name="c", subcore_axis_name="s", num_cores=NC,
                num_subcores=NS)


def _attn_sc(a_src, a_dst, srcf, dstf):
    @functools.partial(
        pl.kernel,
        out_type=(
            jax.ShapeDtypeStruct((NW, 1, EPWP), jnp.float32),
            jax.ShapeDtypeStruct((NW, 1, N_NODES), jnp.float32),
        ),
        mesh=plsc.VectorSubcoreMesh(**_SC_MESH),
        scratch_types=dict(
            src1=pltpu.VMEM((EPWP,), jnp.int32),
            dst1=pltpu.VMEM((EPWP,), jnp.int32),
            asv=pltpu.VMEM((N_NODES,), jnp.float32),
            adv=pltpu.VMEM((N_NODES,), jnp.float32),
            wv=pltpu.VMEM((EPWP,), jnp.float32),
            denv=pltpu.VMEM((N_NODES,), jnp.float32),
        ),
        compiler_params=pltpu.CompilerParams(needs_layout_passes=False),
    )
    def k(asrc_hbm, adst_hbm, src_hbm, dst_hbm, w_out, den_out,
          src1, dst1, asv, adv, wv, denv):
        cid = lax.axis_index("c")
        sid = lax.axis_index("s")
        wid = sid * NC + cid

        pltpu.sync_copy(src_hbm.at[wid, 0], src1)
        pltpu.sync_copy(dst_hbm.at[wid, 0], dst1)
        pltpu.sync_copy(asrc_hbm, asv)
        pltpu.sync_copy(adst_hbm, adv)

        zeros16 = jnp.zeros((16,), jnp.float32)

        def zden(i, _):
            for t in range(5):
                denv[pl.ds(i * 80 + t * 16, 16)] = zeros16
            return 0
        lax.fori_loop(0, N_NODES // 80, zden, 0)

        def edge16(e0):
            sv = src1[pl.ds(e0, 16)]
            dv = dst1[pl.ds(e0, 16)]
            a = plsc.load_gather(asv, [sv]) + plsc.load_gather(adv, [dv])
            a = jnp.where(a > 0, a, a * 0.2)
            w = jnp.exp(a)
            wv[pl.ds(e0, 16)] = w
            plsc.addupdate_scatter(denv, [dv], w)

        def edge32(j, _):
            edge16(j * 32)
            edge16(j * 32 + 16)
            return 0
        lax.fori_loop(0, EPW // 32, edge32, 0)
        edge16(EPW - 16)

        for t in range((EPWP - EPW) // 16):
            wv[pl.ds(EPW + t * 16, 16)] = zeros16

        pltpu.sync_copy(wv, w_out.at[wid, 0])
        pltpu.sync_copy(denv, den_out.at[wid, 0])

    w3, den = k(a_src, a_dst, srcf, dstf)
    return w3, den.reshape(NW, N_NODES)


def _agg_sc(h, w4, srcf, dst4):
    @functools.partial(
        pl.kernel,
        out_type=jax.ShapeDtypeStruct((NC, N_NODES, D), jnp.float32),
        mesh=plsc.VectorSubcoreMesh(**_SC_MESH),
        scratch_types=dict(
            src1=pltpu.VMEM((EPWP,), jnp.int32),
            db2=pltpu.VMEM((2, CBH), jnp.int32),
            wb2=pltpu.VMEM((2, CBH), jnp.float32),
            r0=pltpu.VMEM((CBH, D // 2), jnp.float32),
            r1=pltpu.VMEM((CBH, D // 2), jnp.float32),
            ro0=pltpu.VMEM((CBH, D), jnp.float32),
            ro1=pltpu.VMEM((CBH, D), jnp.float32),
            acc_s=pltpu.VMEM_SHARED((N_NODES, D), jnp.float32),
            sdw=pltpu.SemaphoreType.DMA,
            sg0=pltpu.SemaphoreType.DMA,
            sg1=pltpu.SemaphoreType.DMA,
            ss0=pltpu.SemaphoreType.DMA,
            ss1=pltpu.SemaphoreType.DMA,
        ),
        compiler_params=pltpu.CompilerParams(needs_layout_passes=False, use_tc_tiling_on_sc=False),
    )
    def k(h_hbm, w_hbm, src_hbm, dst_hbm, acc_out,
          src1, db2, wb2, r0, r1, ro0, ro1, acc_s, sdw, sg0, sg1, ss0, ss1):
        cid = lax.axis_index("c")
        sid = lax.axis_index("s")
        wid = sid * NC + cid

        pltpu.sync_copy(src_hbm.at[wid, 0], src1)

        zeros16 = jnp.zeros((16,), jnp.float32)

        for i in range(16):
            for t in range(D // 16):
                ro0[i, pl.ds(t * 16, 16)] = zeros16

        def zacc(i, _):
            pltpu.sync_copy(ro0.at[pl.ds(0, 16)],
                            acc_s.at[pl.ds(sid * RPS + i * 16, 16)])
            return 0
        lax.fori_loop(0, RPS // 16, zacc, 0)

        @pl.when(sid == NS - 1)
        def _():
            pltpu.sync_copy(ro0.at[pl.ds(0, 16)], acc_s.at[pl.ds(NS * RPS, TAIL)])

        plsc.subcore_barrier()

        def scale(rp, ro, half):
            for g in range(CBH // 16):
                wv16 = wb2[half, pl.ds(g * 16, 16)]
                for r in range(16):
                    ws = jnp.full((16,), wv16[r], jnp.float32)
                    wsb = plsc.pack(ws, ws, format=plsc.PackFormat.INTERLEAVED)
                    row = g * 16 + r
                    for t in range(D // 32):
                        v = rp[row, pl.ds(t * 16, 16)]
                        p = plsc.bitcast(v, jnp.bfloat16) * wsb
                        a, b = plsc.unpack(p, format=plsc.PackFormat.INTERLEAVED)
                        ro[row, pl.ds(t * 32, 16)] = a
                        ro[row, pl.ds(t * 32 + 16, 16)] = b

        def wait_g(rbuf, sem):
            pltpu.make_async_copy(h_hbm.at[pl.ds(0, CBH)], rbuf, sem).wait()

        pltpu.async_copy(dst_hbm.at[wid, 0], db2, sdw)
        pltpu.async_copy(w_hbm.at[wid, 0], wb2, sdw)
        pltpu.async_copy(h_hbm.at[src1.at[pl.ds(0, CBH)]], r0, sg0)
        pltpu.async_copy(h_hbm.at[src1.at[pl.ds(CBH, CBH)]], r1, sg1)

        def pair_body(t, _):
            base = t * 2 * CBH
            wait_g(r0, sg0)
            pltpu.make_async_copy(dst_hbm.at[wid, 0], db2, sdw).wait()
            pltpu.make_async_copy(w_hbm.at[wid, 0], wb2, sdw).wait()
            scale(r0, ro0, 0)
            s0 = pltpu.async_copy(ro0, acc_s.at[db2.at[0]], ss0, add=True)

            @pl.when(t < PAIRS - 1)
            def _():
                pltpu.async_copy(
                    h_hbm.at[src1.at[pl.ds(base + 2 * CBH, CBH)]], r0, sg0)

            wait_g(r1, sg1)
            scale(r1, ro1, 1)
            s1 = pltpu.async_copy(ro1, acc_s.at[db2.at[1]], ss1, add=True)

            @pl.when(t < PAIRS - 1)
            def _():
                pltpu.async_copy(
                    h_hbm.at[src1.at[pl.ds(base + 3 * CBH, CBH)]], r1, sg1)

            s0.wait()
            s1.wait()

            @pl.when(t < PAIRS - 1)
            def _():
                pltpu.async_copy(dst_hbm.at[wid, t + 1], db2, sdw)
                pltpu.async_copy(w_hbm.at[wid, t + 1], wb2, sdw)
            return 0
        lax.fori_loop(0, PAIRS, pair_body, 0)

        plsc.subcore_barrier()
        pltpu.sync_copy(acc_s.at[pl.ds(sid * RPS, RPS)],
                        acc_out.at[cid].at[pl.ds(sid * RPS, RPS)])

        @pl.when(sid == NS - 1)
        def _():
            pltpu.sync_copy(acc_s.at[pl.ds(NS * RPS, TAIL)],
                            acc_out.at[cid].at[pl.ds(NS * RPS, TAIL)])

    return k(h, w4, srcf, dst4)


def _pack_h(h):
    n = h.shape[0]
    hb = h.astype(jnp.bfloat16).reshape(n, D // 32, 2, 16)
    st = jnp.stack([hb[:, :, 0, :], hb[:, :, 1, :]], axis=-1)
    return lax.bitcast_convert_type(st, jnp.float32).reshape(n, D // 2)


def _edge_sc(h, a_src, a_dst, srcf, dstf, dst4):
    w3, den32 = _attn_sc(a_src, a_dst, srcf, dstf)
    w4 = w3.reshape(NW, PAIRS, 2, CBH)
    acc2 = _agg_sc(_pack_h(h), w4, srcf, dst4)
    return acc2, den32


def _attmat(att_src, att_dst):
    A = jnp.zeros((D, D), jnp.float32)
    return A.at[:, 0].set(att_src).at[:, 1].set(att_dst)


def kernel(x, edge_index, W1, att_src1, att_dst1, b1, W2, att_src2, att_dst2, b2):
    pad = ((0, 0), (0, EPWP - EPW))
    srcf = jnp.pad(edge_index[0].reshape(NW, EPW), pad).reshape(NW, 1, EPWP)
    dstf = jnp.pad(edge_index[1].reshape(NW, EPW), pad).reshape(NW, 1, EPWP)
    dst4 = dstf.reshape(NW, PAIRS, 2, CBH)

    h1, aux1 = _mm(x, W1, _attmat(att_src1, att_dst1))
    acc1, den1 = _edge_sc(h1, aux1[:, 0], aux1[:, 1], srcf, dstf, dst4)

    h2, aux2 = _norm_mm(acc1, den1, b1, W2, _attmat(att_src2, att_dst2))
    acc2, den2 = _edge_sc(h2, aux2[:, 0], aux2[:, 1], srcf, dstf, dst4)

    return _norm(acc2, den2, b2)
